# baseline (speedup 1.0000x reference)
import functools

import jax
import jax.numpy as jnp
from jax import lax
from jax.experimental import pallas as pl
from jax.experimental.pallas import tpu as pltpu
from jax.experimental.pallas import tpu_sc as plsc

NG = 100
F = 128
N = NG * F
E = 409600
HID = 64
NGP = 104
TRI = F * (F - 1) // 2
HIDDEN = 128

NC = 2
NS = 16
NW = NC * NS
EPT = E // NW
CH = 128
NCH = EPT // CH
RPT = N // NS
NBUF = 5

_HI = lax.Precision.HIGHEST


def _dot(a, b):
  return lax.dot_general(a, b, (((1,), (0,)), ((), ())))


def _deg_body(d_hbm, zeros_hbm, ones_hbm, out_hbm, zbuf, ones_v, didx, hist,
              *sems):
  c = lax.axis_index("c")
  t = lax.axis_index("s")

  pltpu.sync_copy(zeros_hbm, zbuf)
  pltpu.sync_copy(ones_hbm, ones_v)
  crow = (c * NS + t) * NCH
  pltpu.sync_copy(d_hbm.at[pl.ds(crow, NCH)], didx)

  row0 = t * RPT
  pltpu.sync_copy(zbuf, hist.at[pl.ds(row0, RPT)])
  plsc.subcore_barrier()

  def body(m, carry):
    descs = []
    for b in range(NBUF):
      descs.append(
          pltpu.async_copy(ones_v, hist.at[didx.at[m * NBUF + b]], sems[b],
                           add=True))
    for d in descs:
      d.wait()
    return carry
  lax.fori_loop(0, NCH // NBUF, body, 0)

  plsc.subcore_barrier()
  pltpu.sync_copy(hist.at[pl.ds(row0, RPT)], zbuf)
  pltpu.sync_copy(zbuf, out_hbm.at[c, pl.ds(row0, RPT)])


def _edge_body(hwp_hbm, s_hbm, d_hbm, zeros_hbm, out_hbm, zbuf, rows, sidx,
               didx, agg, *sems):
  c = lax.axis_index("c")
  t = lax.axis_index("s")

  pltpu.sync_copy(zeros_hbm, zbuf)
  crow = (c * NS + t) * NCH
  pltpu.sync_copy(s_hbm.at[pl.ds(crow, NCH)], sidx)
  pltpu.sync_copy(d_hbm.at[pl.ds(crow, NCH)], didx)

  row0 = t * RPT
  for p in range(RPT // 200):
    pltpu.sync_copy(zbuf, agg.at[pl.ds(row0 + p * 200, 200)])
  plsc.subcore_barrier()

  gsems = sems[:NBUF]
  ssems = sems[NBUF:]

  def body(m, carry):
    gd = []
    for b in range(NBUF):
      gd.append(
          pltpu.async_copy(hwp_hbm.at[sidx.at[m * NBUF + b]], rows.at[b],
                           gsems[b]))
    sd = []
    for b in range(NBUF):
      gd[b].wait()
      sd.append(
          pltpu.async_copy(rows.at[b], agg.at[didx.at[m * NBUF + b]],
                           ssems[b], add=True))
    for d in sd:
      d.wait()
    return carry
  lax.fori_loop(0, NCH // NBUF, body, 0)

  plsc.subcore_barrier()
  for p in range(RPT // 200):
    pltpu.sync_copy(agg.at[pl.ds(row0 + p * 200, 200)], zbuf)
    pltpu.sync_copy(zbuf, out_hbm.at[c, pl.ds(row0 + p * 200, 200)])


@functools.cache
def _sc_kernels():
  mesh = plsc.VectorSubcoreMesh(
      core_axis_name="c", subcore_axis_name="s",
      num_cores=NC, num_subcores=NS)
  params = pltpu.CompilerParams(use_tc_tiling_on_sc=False)
  deg = pl.kernel(
      _deg_body,
      compiler_params=params,
      out_type=jax.ShapeDtypeStruct((NC, N, 8), jnp.float32),
      mesh=mesh,
      scratch_types=[
          pltpu.VMEM((RPT, 8), jnp.float32),
          pltpu.VMEM((CH, 8), jnp.float32),
          pltpu.VMEM((NCH, CH), jnp.int32),
          pltpu.VMEM_SHARED((N, 8), jnp.float32),
      ] + [pltpu.SemaphoreType.DMA] * NBUF)
  edge = pl.kernel(
      _edge_body,
      compiler_params=params,
      out_type=jax.ShapeDtypeStruct((NC, N, HID), jnp.float32),
      mesh=mesh,
      scratch_types=[
          pltpu.VMEM((200, HID), jnp.float32),
          pltpu.VMEM((NBUF, CH, HID), jnp.float32),
          pltpu.VMEM((NCH, CH), jnp.int32),
          pltpu.VMEM((NCH, CH), jnp.int32),
          pltpu.VMEM_SHARED((N, HID), jnp.float32),
      ] + [pltpu.SemaphoreType.DMA] * (2 * NBUF))
  return deg, edge


GB = 10
RB = GB * F


def _prepa_body(x_ref, w0_ref, hw_ref):
  hw_ref[...] = _dot(x_ref[...], w0_ref[...])


def _prepb_body(hw_ref, p_ref, hwp_ref, dinv_ref):
  deg = p_ref[0, :, 0:1] + p_ref[1, :, 0:1] + 1.0
  dinv = lax.rsqrt(deg)
  hwp_ref[...] = hw_ref[...] * dinv
  dinv_ref[...] = jnp.broadcast_to(dinv, (RB, 8))


def _mid_body(p_ref, hwp_ref, dinv_ref, b_ref, w_ref, hwp1_ref, m_ref):
  dinv = dinv_ref[:, 0:1]
  x1 = jnp.tanh(dinv * (p_ref[0] + p_ref[1] + hwp_ref[...]) + b_ref[...])
  m_ref[...] = jnp.sum(x1.reshape(GB, F, HID), axis=1)[None] * (1.0 / F)
  hwp1_ref[...] = _dot(x1, w_ref[...]) * dinv


def _expand_body(xf_ref, w8_ref, g_ref, b_ref, wf_ref, m_ref, rg_ref, be_ref,
                 ge_ref):
  xp = xf_ref[...]
  inv = 1.0 / NG
  m = jnp.sum(xp, axis=0, keepdims=True) * inv
  var = jnp.sum(xp * xp, axis=0, keepdims=True) * inv - m * m
  r = lax.rsqrt(var + 1e-5)

  ge_ref[...] = jnp.zeros((1, F * F), jnp.float32)
  be_ref[...] = jnp.zeros((1, F * F), jnp.float32)
  off = 0
  for i in range(F - 1):
    seg = F - 1 - i
    ge_ref[0, pl.ds(i * F + i + 1, seg)] = g_ref[0, pl.ds(off, seg)]
    be_ref[0, pl.ds(i * F + i + 1, seg)] = b_ref[0, pl.ds(off, seg)]
    off += seg
  m_ref[...] = m
  rg_ref[...] = r * ge_ref[...]

  wf_ref[...] = jnp.zeros((F * F, HIDDEN), jnp.float32)
  off = 0
  for i in range(F - 1):
    seg = F - 1 - i
    wf_ref[pl.ds(i * F + i + 1, seg), :] = w8_ref[pl.ds(off, seg), :]
    off += seg


def _tail_body(q, hwp1, dinv8, b1c, xflat, wf, mf, rgf, bef, m1, bnhg, bnhb,
               w0b, b0, g0, be0, w1, b1m, g1, be1, w2, b2m, g2, be2,
               w3, b3m, out):
  dinv = dinv8[:, 0:1]
  x2 = jnp.tanh(dinv * (q[0] + q[1] + hwp1[...]) + b1c[...])
  m2 = jnp.sum(x2.reshape(NG, F, HID), axis=1) * (1.0 / F)

  rows = lax.broadcasted_iota(jnp.int32, (NGP, 1), 0)
  mask = rows < NG
  inv = 1.0 / NG

  def stats(vm):
    m = jnp.sum(vm, axis=0, keepdims=True) * inv
    var = jnp.sum(vm * vm, axis=0, keepdims=True) * inv - m * m
    return m, lax.rsqrt(var + 1e-5)

  zp = jnp.zeros((NGP - NG, HID), jnp.float32)
  h = jnp.concatenate([
      jnp.concatenate([m1[...], zp], axis=0),
      jnp.concatenate([m2, zp], axis=0)], axis=1)
  mh, rh = stats(h)
  hn = (h - mh) * rh * bnhg[...] + bnhb[...]

  def bstage(z, g, b):
    zm = jnp.where(mask, z, 0.0)
    mz, rz = stats(zm)
    return jax.nn.relu((z - mz) * rz * g[...] + b[...])

  vn = (xflat[...] - mf[...]) * rgf[...] + bef[...]
  z = bstage(
      _dot(vn, wf[...]) + _dot(hn, w0b[...]) + b0[...],
      g0, be0)
  z = bstage(_dot(z, w1[...]) + b1m[...], g1, be1)
  z = bstage(_dot(z, w2[...]) + b2m[...], g2, be2)
  out[...] = _dot(z, w3[...]) + b3m[...]


_prepa = pl.pallas_call(
    _prepa_body,
    grid=(N // RB,),
    in_specs=[
        pl.BlockSpec((RB, F), lambda i: (i, 0)),
        pl.BlockSpec((F, HID), lambda i: (0, 0)),
    ],
    out_specs=pl.BlockSpec((RB, HID), lambda i: (i, 0)),
    out_shape=jax.ShapeDtypeStruct((N, HID), jnp.float32),
)

_prepb = pl.pallas_call(
    _prepb_body,
    grid=(N // RB,),
    in_specs=[
        pl.BlockSpec((RB, HID), lambda i: (i, 0)),
        pl.BlockSpec((NC, RB, 8), lambda i: (0, i, 0)),
    ],
    out_specs=[
        pl.BlockSpec((RB, HID), lambda i: (i, 0)),
        pl.BlockSpec((RB, 8), lambda i: (i, 0)),
    ],
    out_shape=[
        jax.ShapeDtypeStruct((N, HID), jnp.float32),
        jax.ShapeDtypeStruct((N, 8), jnp.float32),
    ],
)

_mid = pl.pallas_call(
    _mid_body,
    grid=(N // RB,),
    in_specs=[
        pl.BlockSpec((NC, RB, HID), lambda i: (0, i, 0)),
        pl.BlockSpec((RB, HID), lambda i: (i, 0)),
        pl.BlockSpec((RB, 8), lambda i: (i, 0)),
        pl.BlockSpec((1, HID), lambda i: (0, 0)),
        pl.BlockSpec((HID, HID), lambda i: (0, 0)),
    ],
    out_specs=[
        pl.BlockSpec((RB, HID), lambda i: (i, 0)),
        pl.BlockSpec((1, GB, HID), lambda i: (i, 0, 0)),
    ],
    out_shape=[
        jax.ShapeDtypeStruct((N, HID), jnp.float32),
        jax.ShapeDtypeStruct((NG // GB, GB, HID), jnp.float32),
    ],
)

_expand = pl.pallas_call(
    _expand_body,
    out_shape=[
        jax.ShapeDtypeStruct((F * F, HIDDEN), jnp.float32),
        jax.ShapeDtypeStruct((1, F * F), jnp.float32),
        jax.ShapeDtypeStruct((1, F * F), jnp.float32),
        jax.ShapeDtypeStruct((1, F * F), jnp.float32),
    ],
    scratch_shapes=[
        pltpu.VMEM((1, F * F), jnp.float32),
    ],
)

_tail = pl.pallas_call(
    _tail_body,
    out_shape=jax.ShapeDtypeStruct((NGP, 2), jnp.float32),
)


def kernel(x, edge_index, batch, params):
  del batch
  src = edge_index[0]
  dst = edge_index[1]

  deg_k, edge_k = _sc_kernels()
  src = src.reshape(E // CH, CH)
  dst = dst.reshape(E // CH, CH)
  zeros8 = jnp.zeros((RPT, 8), jnp.float32)
  ones8 = jnp.ones((CH, 8), jnp.float32)
  zeros64 = jnp.zeros((200, HID), jnp.float32)
  r = lambda v: v.reshape(1, -1)
  degp = deg_k(dst, zeros8, ones8)
  hw0 = _prepa(x, params["conv0_W"])
  xflat = jnp.pad(x.reshape(NG, F * F), ((0, NGP - NG), (0, 0)))
  wf, mf, rgf, bef = _expand(xflat, params["mlp0_W"][:TRI],
                             r(params["bn_g"]), r(params["bn_b"]))

  hwp0, dinv8 = _prepb(hw0, degp)
  agg0 = edge_k(hwp0, src, dst, zeros64)
  hwp1, m1 = _mid(agg0, hwp0, dinv8, params["conv0_b"].reshape(1, HID),
                  params["conv1_W"])
  agg1 = edge_k(hwp1, src, dst, zeros64)

  out = _tail(
      agg1, hwp1, dinv8, params["conv1_b"].reshape(1, HID),
      xflat, wf, mf, rgf, bef, m1.reshape(NG, HID),
      r(params["bnh_g"]), r(params["bnh_b"]),
      params["mlp0_W"][TRI:], r(params["mlp0_b"]),
      r(params["mbn0_g"]), r(params["mbn0_b"]),
      params["mlp1_W"], r(params["mlp1_b"]),
      r(params["mbn1_g"]), r(params["mbn1_b"]),
      params["mlp2_W"], r(params["mlp2_b"]),
      r(params["mbn2_g"]), r(params["mbn2_b"]),
      params["mlp3_W"], r(params["mlp3_b"]))
  return out[:NG]


# speedup vs baseline: 1.1290x; 1.1290x over previous
import functools

import jax
import jax.numpy as jnp
from jax import lax
from jax.experimental import pallas as pl
from jax.experimental.pallas import tpu as pltpu
from jax.experimental.pallas import tpu_sc as plsc

NG = 100
F = 128
N = NG * F
E = 409600
HID = 64
NGP = 104
TRI = F * (F - 1) // 2
HIDDEN = 128

NC = 2
NS = 16
NW = NC * NS
EPT = E // NW
CH = 128
NCH = EPT // CH
RPT = N // NS
NBUF = 5

_HI = lax.Precision.HIGHEST


def _dot(a, b):
  return lax.dot_general(a, b, (((1,), (0,)), ((), ())))


def _deg_body(d_hbm, zeros_hbm, ones_hbm, out_hbm, zbuf, ones_v, didx, hist,
              *sems):
  c = lax.axis_index("c")
  t = lax.axis_index("s")

  pltpu.sync_copy(zeros_hbm, zbuf)
  pltpu.sync_copy(ones_hbm, ones_v)
  crow = (c * NS + t) * NCH
  pltpu.sync_copy(d_hbm.at[pl.ds(crow, NCH)], didx)

  row0 = t * RPT
  pltpu.sync_copy(zbuf, hist.at[pl.ds(row0, RPT)])
  plsc.subcore_barrier()

  def body(m, carry):
    descs = []
    for b in range(NBUF):
      descs.append(
          pltpu.async_copy(ones_v, hist.at[didx.at[m * NBUF + b]], sems[b],
                           add=True))
    for d in descs:
      d.wait()
    return carry
  lax.fori_loop(0, NCH // NBUF, body, 0)

  plsc.subcore_barrier()
  pltpu.sync_copy(hist.at[pl.ds(row0, RPT)], zbuf)
  pltpu.sync_copy(zbuf, out_hbm.at[c, pl.ds(row0, RPT)])


def _edge_body(hwp_hbm, s_hbm, d_hbm, zeros_hbm, out_hbm, zbuf, rows, sidx,
               didx, agg, *sems):
  c = lax.axis_index("c")
  t = lax.axis_index("s")

  pltpu.sync_copy(zeros_hbm, zbuf)
  crow = (c * NS + t) * NCH
  pltpu.sync_copy(s_hbm.at[pl.ds(crow, NCH)], sidx)
  pltpu.sync_copy(d_hbm.at[pl.ds(crow, NCH)], didx)

  row0 = t * RPT
  for p in range(RPT // 200):
    pltpu.sync_copy(zbuf, agg.at[pl.ds(row0 + p * 200, 200)])
  plsc.subcore_barrier()

  gsems = sems[:NBUF]
  ssems = sems[NBUF:]

  def body(m, carry):
    gd = []
    for b in range(NBUF):
      @pl.when(m > 0)
      def _drain():
        pltpu.make_async_copy(rows.at[b], agg.at[didx.at[0]],
                              ssems[b]).wait()
      gd.append(
          pltpu.async_copy(hwp_hbm.at[sidx.at[m * NBUF + b]], rows.at[b],
                           gsems[b]))
    for b in range(NBUF):
      gd[b].wait()
      pltpu.async_copy(rows.at[b], agg.at[didx.at[m * NBUF + b]],
                       ssems[b], add=True)
    return carry
  lax.fori_loop(0, NCH // NBUF, body, 0)
  for b in range(NBUF):
    pltpu.make_async_copy(rows.at[b], agg.at[didx.at[0]], ssems[b]).wait()

  plsc.subcore_barrier()
  for p in range(RPT // 200):
    pltpu.sync_copy(agg.at[pl.ds(row0 + p * 200, 200)], zbuf)
    pltpu.sync_copy(zbuf, out_hbm.at[c, pl.ds(row0 + p * 200, 200)])


@functools.cache
def _sc_kernels():
  mesh = plsc.VectorSubcoreMesh(
      core_axis_name="c", subcore_axis_name="s",
      num_cores=NC, num_subcores=NS)
  params = pltpu.CompilerParams(use_tc_tiling_on_sc=False)
  deg = pl.kernel(
      _deg_body,
      compiler_params=params,
      out_type=jax.ShapeDtypeStruct((NC, N, 8), jnp.float32),
      mesh=mesh,
      scratch_types=[
          pltpu.VMEM((RPT, 8), jnp.float32),
          pltpu.VMEM((CH, 8), jnp.float32),
          pltpu.VMEM((NCH, CH), jnp.int32),
          pltpu.VMEM_SHARED((N, 8), jnp.float32),
      ] + [pltpu.SemaphoreType.DMA] * NBUF)
  edge = pl.kernel(
      _edge_body,
      compiler_params=params,
      out_type=jax.ShapeDtypeStruct((NC, N, HID), jnp.float32),
      mesh=mesh,
      scratch_types=[
          pltpu.VMEM((200, HID), jnp.float32),
          pltpu.VMEM((NBUF, CH, HID), jnp.float32),
          pltpu.VMEM((NCH, CH), jnp.int32),
          pltpu.VMEM((NCH, CH), jnp.int32),
          pltpu.VMEM_SHARED((N, HID), jnp.float32),
      ] + [pltpu.SemaphoreType.DMA] * (2 * NBUF))
  return deg, edge


GB = 50
RB = GB * F


def _prepa_body(x_ref, w0_ref, hw_ref):
  hw_ref[...] = _dot(x_ref[...], w0_ref[...])


def _prepb_body(hw_ref, p_ref, hwp_ref, dinv_ref):
  deg = p_ref[0, :, 0:1] + p_ref[1, :, 0:1] + 1.0
  dinv = lax.rsqrt(deg)
  hwp_ref[...] = hw_ref[...] * dinv
  dinv_ref[...] = jnp.broadcast_to(dinv, (RB, 8))


def _mid_body(p_ref, hwp_ref, dinv_ref, b_ref, w_ref, hwp1_ref, m_ref):
  dinv = dinv_ref[:, 0:1]
  x1 = jnp.tanh(dinv * (p_ref[0] + p_ref[1] + hwp_ref[...]) + b_ref[...])
  m_ref[...] = jnp.sum(x1.reshape(GB, F, HID), axis=1)[None] * (1.0 / F)
  hwp1_ref[...] = _dot(x1, w_ref[...]) * dinv


def _expand_body(xf_ref, w8_ref, g_ref, b_ref, wf_ref, m_ref, rg_ref, be_ref,
                 ge_ref):
  xp = xf_ref[...]
  inv = 1.0 / NG
  m = jnp.sum(xp, axis=0, keepdims=True) * inv
  var = jnp.sum(xp * xp, axis=0, keepdims=True) * inv - m * m
  r = lax.rsqrt(var + 1e-5)

  ge_ref[...] = jnp.zeros((1, F * F), jnp.float32)
  be_ref[...] = jnp.zeros((1, F * F), jnp.float32)
  off = 0
  for i in range(F - 1):
    seg = F - 1 - i
    ge_ref[0, pl.ds(i * F + i + 1, seg)] = g_ref[0, pl.ds(off, seg)]
    be_ref[0, pl.ds(i * F + i + 1, seg)] = b_ref[0, pl.ds(off, seg)]
    off += seg
  m_ref[...] = m
  rg_ref[...] = r * ge_ref[...]

  wf_ref[...] = jnp.zeros((F * F, HIDDEN), jnp.float32)
  off = 0
  for i in range(F - 1):
    seg = F - 1 - i
    wf_ref[pl.ds(i * F + i + 1, seg), :] = w8_ref[pl.ds(off, seg), :]
    off += seg


def _tail_body(q, hwp1, dinv8, b1c, xflat, wf, mf, rgf, bef, m1, bnhg, bnhb,
               w0b, b0, g0, be0, w1, b1m, g1, be1, w2, b2m, g2, be2,
               w3, b3m, out):
  dinv = dinv8[:, 0:1]
  x2 = jnp.tanh(dinv * (q[0] + q[1] + hwp1[...]) + b1c[...])
  m2 = jnp.sum(x2.reshape(NG, F, HID), axis=1) * (1.0 / F)

  rows = lax.broadcasted_iota(jnp.int32, (NGP, 1), 0)
  mask = rows < NG
  inv = 1.0 / NG

  def stats(vm):
    m = jnp.sum(vm, axis=0, keepdims=True) * inv
    var = jnp.sum(vm * vm, axis=0, keepdims=True) * inv - m * m
    return m, lax.rsqrt(var + 1e-5)

  zp = jnp.zeros((NGP - NG, HID), jnp.float32)
  h = jnp.concatenate([
      jnp.concatenate([m1[...], zp], axis=0),
      jnp.concatenate([m2, zp], axis=0)], axis=1)
  mh, rh = stats(h)
  hn = (h - mh) * rh * bnhg[...] + bnhb[...]

  def bstage(z, g, b):
    zm = jnp.where(mask, z, 0.0)
    mz, rz = stats(zm)
    return jax.nn.relu((z - mz) * rz * g[...] + b[...])

  vn = (xflat[...] - mf[...]) * rgf[...] + bef[...]
  z = bstage(
      _dot(vn, wf[...]) + _dot(hn, w0b[...]) + b0[...],
      g0, be0)
  z = bstage(_dot(z, w1[...]) + b1m[...], g1, be1)
  z = bstage(_dot(z, w2[...]) + b2m[...], g2, be2)
  out[...] = _dot(z, w3[...]) + b3m[...]


_prepa = pl.pallas_call(
    _prepa_body,
    grid=(N // RB,),
    in_specs=[
        pl.BlockSpec((RB, F), lambda i: (i, 0)),
        pl.BlockSpec((F, HID), lambda i: (0, 0)),
    ],
    out_specs=pl.BlockSpec((RB, HID), lambda i: (i, 0)),
    out_shape=jax.ShapeDtypeStruct((N, HID), jnp.float32),
)

_prepb = pl.pallas_call(
    _prepb_body,
    grid=(N // RB,),
    in_specs=[
        pl.BlockSpec((RB, HID), lambda i: (i, 0)),
        pl.BlockSpec((NC, RB, 8), lambda i: (0, i, 0)),
    ],
    out_specs=[
        pl.BlockSpec((RB, HID), lambda i: (i, 0)),
        pl.BlockSpec((RB, 8), lambda i: (i, 0)),
    ],
    out_shape=[
        jax.ShapeDtypeStruct((N, HID), jnp.float32),
        jax.ShapeDtypeStruct((N, 8), jnp.float32),
    ],
)

_mid = pl.pallas_call(
    _mid_body,
    grid=(N // RB,),
    in_specs=[
        pl.BlockSpec((NC, RB, HID), lambda i: (0, i, 0)),
        pl.BlockSpec((RB, HID), lambda i: (i, 0)),
        pl.BlockSpec((RB, 8), lambda i: (i, 0)),
        pl.BlockSpec((1, HID), lambda i: (0, 0)),
        pl.BlockSpec((HID, HID), lambda i: (0, 0)),
    ],
    out_specs=[
        pl.BlockSpec((RB, HID), lambda i: (i, 0)),
        pl.BlockSpec((1, GB, HID), lambda i: (i, 0, 0)),
    ],
    out_shape=[
        jax.ShapeDtypeStruct((N, HID), jnp.float32),
        jax.ShapeDtypeStruct((NG // GB, GB, HID), jnp.float32),
    ],
)

_expand = pl.pallas_call(
    _expand_body,
    out_shape=[
        jax.ShapeDtypeStruct((F * F, HIDDEN), jnp.float32),
        jax.ShapeDtypeStruct((1, F * F), jnp.float32),
        jax.ShapeDtypeStruct((1, F * F), jnp.float32),
        jax.ShapeDtypeStruct((1, F * F), jnp.float32),
    ],
    scratch_shapes=[
        pltpu.VMEM((1, F * F), jnp.float32),
    ],
)

_tail = pl.pallas_call(
    _tail_body,
    out_shape=jax.ShapeDtypeStruct((NGP, 2), jnp.float32),
)


def kernel(x, edge_index, batch, params):
  del batch
  src = edge_index[0]
  dst = edge_index[1]

  deg_k, edge_k = _sc_kernels()
  src = src.reshape(E // CH, CH)
  dst = dst.reshape(E // CH, CH)
  zeros8 = jnp.zeros((RPT, 8), jnp.float32)
  ones8 = jnp.ones((CH, 8), jnp.float32)
  zeros64 = jnp.zeros((200, HID), jnp.float32)
  r = lambda v: v.reshape(1, -1)
  degp = deg_k(dst, zeros8, ones8)
  hw0 = _prepa(x, params["conv0_W"])
  xflat = jnp.pad(x.reshape(NG, F * F), ((0, NGP - NG), (0, 0)))
  wf, mf, rgf, bef = _expand(xflat, params["mlp0_W"][:TRI],
                             r(params["bn_g"]), r(params["bn_b"]))

  hwp0, dinv8 = _prepb(hw0, degp)
  agg0 = edge_k(hwp0, src, dst, zeros64)
  hwp1, m1 = _mid(agg0, hwp0, dinv8, params["conv0_b"].reshape(1, HID),
                  params["conv1_W"])
  agg1 = edge_k(hwp1, src, dst, zeros64)

  out = _tail(
      agg1, hwp1, dinv8, params["conv1_b"].reshape(1, HID),
      xflat, wf, mf, rgf, bef, m1.reshape(NG, HID),
      r(params["bnh_g"]), r(params["bnh_b"]),
      params["mlp0_W"][TRI:], r(params["mlp0_b"]),
      r(params["mbn0_g"]), r(params["mbn0_b"]),
      params["mlp1_W"], r(params["mlp1_b"]),
      r(params["mbn1_g"]), r(params["mbn1_b"]),
      params["mlp2_W"], r(params["mlp2_b"]),
      r(params["mbn2_g"]), r(params["mbn2_b"]),
      params["mlp3_W"], r(params["mlp3_b"]))
  return out[:NG]


# speedup vs baseline: 1.1976x; 1.0607x over previous
import functools

import jax
import jax.numpy as jnp
from jax import lax
from jax.experimental import pallas as pl
from jax.experimental.pallas import tpu as pltpu
from jax.experimental.pallas import tpu_sc as plsc

NG = 100
F = 128
N = NG * F
E = 409600
HID = 64
NGP = 104
TRI = F * (F - 1) // 2
HIDDEN = 128

NC = 2
NS = 16
NW = NC * NS
EPT = E // NW
CH = 128
NCH = EPT // CH
RPT = N // NS
NBUF = 5

_HI = lax.Precision.HIGHEST


def _dot(a, b):
  return lax.dot_general(a, b, (((1,), (0,)), ((), ())))


def _deg_body(d_hbm, zeros_hbm, ones_hbm, out_hbm, zbuf, ones_v, didx, obuf,
              hist, *sems):
  c = lax.axis_index("c")
  t = lax.axis_index("s")

  pltpu.sync_copy(zeros_hbm, zbuf)
  pltpu.sync_copy(ones_hbm, ones_v)
  crow = (c * NS + t) * NCH
  pltpu.sync_copy(d_hbm.at[pl.ds(crow, NCH)], didx)

  row0 = t * RPT
  pltpu.sync_copy(zbuf, hist.at[pl.ds(row0, RPT)])
  plsc.subcore_barrier()

  def body(m, carry):
    descs = []
    for b in range(NBUF):
      descs.append(
          pltpu.async_copy(ones_v, hist.at[didx.at[m * NBUF + b]], sems[b],
                           add=True))
    for d in descs:
      d.wait()
    return carry
  lax.fori_loop(0, NCH // NBUF, body, 0)

  plsc.subcore_barrier()
  pltpu.sync_copy(hist.at[pl.ds(row0, RPT)], zbuf)
  col0 = jnp.zeros((16,), jnp.int32)
  lanes = lax.iota(jnp.int32, 16)
  for grp in range(RPT // 16):
    obuf[pl.ds(grp * 16, 16)] = plsc.load_gather(
        zbuf, [lanes + grp * 16, col0])
  pltpu.sync_copy(obuf, out_hbm.at[c, pl.ds(row0, RPT)])


def _edge_body(hwp_hbm, s_hbm, d_hbm, zeros_hbm, out_hbm, zbuf, rows, sidx,
               didx, agg, *sems):
  c = lax.axis_index("c")
  t = lax.axis_index("s")

  pltpu.sync_copy(zeros_hbm, zbuf)
  crow = (c * NS + t) * NCH
  pltpu.sync_copy(s_hbm.at[pl.ds(crow, NCH)], sidx)
  pltpu.sync_copy(d_hbm.at[pl.ds(crow, NCH)], didx)

  row0 = t * RPT
  for p in range(RPT // 200):
    pltpu.sync_copy(zbuf, agg.at[pl.ds(row0 + p * 200, 200)])
  plsc.subcore_barrier()

  gsems = sems[:NBUF]
  ssems = sems[NBUF:]

  def body(m, carry):
    gd = []
    for b in range(NBUF):
      @pl.when(m > 0)
      def _drain():
        pltpu.make_async_copy(rows.at[b], agg.at[didx.at[0]],
                              ssems[b]).wait()
      gd.append(
          pltpu.async_copy(hwp_hbm.at[sidx.at[m * NBUF + b]], rows.at[b],
                           gsems[b]))
    for b in range(NBUF):
      gd[b].wait()
      pltpu.async_copy(rows.at[b], agg.at[didx.at[m * NBUF + b]],
                       ssems[b], add=True)
    return carry
  lax.fori_loop(0, NCH // NBUF, body, 0)
  for b in range(NBUF):
    pltpu.make_async_copy(rows.at[b], agg.at[didx.at[0]], ssems[b]).wait()

  plsc.subcore_barrier()
  for p in range(RPT // 200):
    pltpu.sync_copy(agg.at[pl.ds(row0 + p * 200, 200)], zbuf)
    pltpu.sync_copy(zbuf, out_hbm.at[c, pl.ds(row0 + p * 200, 200)])


@functools.cache
def _sc_kernels():
  mesh = plsc.VectorSubcoreMesh(
      core_axis_name="c", subcore_axis_name="s",
      num_cores=NC, num_subcores=NS)
  params = pltpu.CompilerParams(use_tc_tiling_on_sc=False)
  deg = pl.kernel(
      _deg_body,
      compiler_params=pltpu.CompilerParams(
          use_tc_tiling_on_sc=False, needs_layout_passes=False),
      out_type=jax.ShapeDtypeStruct((NC, N), jnp.float32),
      mesh=mesh,
      scratch_types=[
          pltpu.VMEM((RPT, 8), jnp.float32),
          pltpu.VMEM((CH, 8), jnp.float32),
          pltpu.VMEM((NCH, CH), jnp.int32),
          pltpu.VMEM((RPT,), jnp.float32),
          pltpu.VMEM_SHARED((N, 8), jnp.float32),
      ] + [pltpu.SemaphoreType.DMA] * NBUF)
  edge = pl.kernel(
      _edge_body,
      compiler_params=params,
      out_type=jax.ShapeDtypeStruct((NC, N, HID), jnp.float32),
      mesh=mesh,
      scratch_types=[
          pltpu.VMEM((200, HID), jnp.float32),
          pltpu.VMEM((NBUF, CH, HID), jnp.float32),
          pltpu.VMEM((NCH, CH), jnp.int32),
          pltpu.VMEM((NCH, CH), jnp.int32),
          pltpu.VMEM_SHARED((N, HID), jnp.float32),
      ] + [pltpu.SemaphoreType.DMA] * (2 * NBUF))
  return deg, edge


GB = 50
RB = GB * F


def _prepa_body(x_ref, w0_ref, hw_ref):
  hw_ref[...] = _dot(x_ref[...], w0_ref[...])


def _prepb_body(hw_ref, p_ref, hwp_ref, dinv_ref):
  deg = (p_ref[0] + p_ref[1] + 1.0)[:, None]
  dinv = lax.rsqrt(deg)
  hwp_ref[...] = hw_ref[...] * dinv
  dinv_ref[...] = jnp.broadcast_to(dinv, (RB, 8))


def _mid_body(p_ref, hwp_ref, dinv_ref, b_ref, w_ref, hwp1_ref, m_ref):
  dinv = dinv_ref[:, 0:1]
  x1 = jnp.tanh(dinv * (p_ref[0] + p_ref[1] + hwp_ref[...]) + b_ref[...])
  m_ref[...] = jnp.sum(x1.reshape(GB, F, HID), axis=1)[None] * (1.0 / F)
  hwp1_ref[...] = _dot(x1, w_ref[...]) * dinv


def _expand_body(xf_ref, w8_ref, g_ref, b_ref, wf_ref, m_ref, rg_ref, be_ref,
                 ge_ref):
  xp = xf_ref[...]
  inv = 1.0 / NG
  m = jnp.sum(xp, axis=0, keepdims=True) * inv
  var = jnp.sum(xp * xp, axis=0, keepdims=True) * inv - m * m
  r = lax.rsqrt(var + 1e-5)

  ge_ref[...] = jnp.zeros((1, F * F), jnp.float32)
  be_ref[...] = jnp.zeros((1, F * F), jnp.float32)
  off = 0
  for i in range(F - 1):
    seg = F - 1 - i
    ge_ref[0, pl.ds(i * F + i + 1, seg)] = g_ref[0, pl.ds(off, seg)]
    be_ref[0, pl.ds(i * F + i + 1, seg)] = b_ref[0, pl.ds(off, seg)]
    off += seg
  m_ref[...] = m
  rg_ref[...] = r * ge_ref[...]

  wf_ref[...] = jnp.zeros((F * F, HIDDEN), jnp.float32)
  off = 0
  for i in range(F - 1):
    seg = F - 1 - i
    wf_ref[pl.ds(i * F + i + 1, seg), :] = w8_ref[pl.ds(off, seg), :]
    off += seg


def _zf_body(xflat, wf, mf, rgf, bef, zf_ref):
  vn = (xflat[...] - mf[...]) * rgf[...] + bef[...]
  zf_ref[...] = _dot(vn, wf[...])


def _tail_body(q, hwp1, dinv8, b1c, zf, m1, bnhg, bnhb,
               w0b, b0, g0, be0, w1, b1m, g1, be1, w2, b2m, g2, be2,
               w3, b3m, out):
  dinv = dinv8[:, 0:1]
  x2 = jnp.tanh(dinv * (q[0] + q[1] + hwp1[...]) + b1c[...])
  m2 = jnp.sum(x2.reshape(NG, F, HID), axis=1) * (1.0 / F)

  rows = lax.broadcasted_iota(jnp.int32, (NGP, 1), 0)
  mask = rows < NG
  inv = 1.0 / NG

  def stats(vm):
    m = jnp.sum(vm, axis=0, keepdims=True) * inv
    var = jnp.sum(vm * vm, axis=0, keepdims=True) * inv - m * m
    return m, lax.rsqrt(var + 1e-5)

  zp = jnp.zeros((NGP - NG, HID), jnp.float32)
  h = jnp.concatenate([
      jnp.concatenate([m1[...], zp], axis=0),
      jnp.concatenate([m2, zp], axis=0)], axis=1)
  mh, rh = stats(h)
  hn = (h - mh) * rh * bnhg[...] + bnhb[...]

  def bstage(z, g, b):
    zm = jnp.where(mask, z, 0.0)
    mz, rz = stats(zm)
    return jax.nn.relu((z - mz) * rz * g[...] + b[...])

  z = bstage(zf[...] + _dot(hn, w0b[...]) + b0[...], g0, be0)
  z = bstage(_dot(z, w1[...]) + b1m[...], g1, be1)
  z = bstage(_dot(z, w2[...]) + b2m[...], g2, be2)
  out[...] = _dot(z, w3[...]) + b3m[...]


_prepa = pl.pallas_call(
    _prepa_body,
    grid=(N // RB,),
    in_specs=[
        pl.BlockSpec((RB, F), lambda i: (i, 0)),
        pl.BlockSpec((F, HID), lambda i: (0, 0)),
    ],
    out_specs=pl.BlockSpec((RB, HID), lambda i: (i, 0)),
    out_shape=jax.ShapeDtypeStruct((N, HID), jnp.float32),
)

_prepb = pl.pallas_call(
    _prepb_body,
    grid=(N // RB,),
    in_specs=[
        pl.BlockSpec((RB, HID), lambda i: (i, 0)),
        pl.BlockSpec((NC, RB), lambda i: (0, i)),
    ],
    out_specs=[
        pl.BlockSpec((RB, HID), lambda i: (i, 0)),
        pl.BlockSpec((RB, 8), lambda i: (i, 0)),
    ],
    out_shape=[
        jax.ShapeDtypeStruct((N, HID), jnp.float32),
        jax.ShapeDtypeStruct((N, 8), jnp.float32),
    ],
)

_mid = pl.pallas_call(
    _mid_body,
    grid=(N // RB,),
    in_specs=[
        pl.BlockSpec((NC, RB, HID), lambda i: (0, i, 0)),
        pl.BlockSpec((RB, HID), lambda i: (i, 0)),
        pl.BlockSpec((RB, 8), lambda i: (i, 0)),
        pl.BlockSpec((1, HID), lambda i: (0, 0)),
        pl.BlockSpec((HID, HID), lambda i: (0, 0)),
    ],
    out_specs=[
        pl.BlockSpec((RB, HID), lambda i: (i, 0)),
        pl.BlockSpec((1, GB, HID), lambda i: (i, 0, 0)),
    ],
    out_shape=[
        jax.ShapeDtypeStruct((N, HID), jnp.float32),
        jax.ShapeDtypeStruct((NG // GB, GB, HID), jnp.float32),
    ],
)

_expand = pl.pallas_call(
    _expand_body,
    out_shape=[
        jax.ShapeDtypeStruct((F * F, HIDDEN), jnp.float32),
        jax.ShapeDtypeStruct((1, F * F), jnp.float32),
        jax.ShapeDtypeStruct((1, F * F), jnp.float32),
        jax.ShapeDtypeStruct((1, F * F), jnp.float32),
    ],
    scratch_shapes=[
        pltpu.VMEM((1, F * F), jnp.float32),
    ],
)

_zf = pl.pallas_call(
    _zf_body,
    out_shape=jax.ShapeDtypeStruct((NGP, HIDDEN), jnp.float32),
)

_tail = pl.pallas_call(
    _tail_body,
    out_shape=jax.ShapeDtypeStruct((NGP, 2), jnp.float32),
)


def kernel(x, edge_index, batch, params):
  del batch
  src = edge_index[0]
  dst = edge_index[1]

  deg_k, edge_k = _sc_kernels()
  src = src.reshape(E // CH, CH)
  dst = dst.reshape(E // CH, CH)
  zeros8 = jnp.zeros((RPT, 8), jnp.float32)
  ones8 = jnp.ones((CH, 8), jnp.float32)
  zeros64 = jnp.zeros((200, HID), jnp.float32)
  r = lambda v: v.reshape(1, -1)
  degp = deg_k(dst, zeros8, ones8)
  hw0 = _prepa(x, params["conv0_W"])
  xflat = jnp.pad(x.reshape(NG, F * F), ((0, NGP - NG), (0, 0)))
  wf, mf, rgf, bef = _expand(xflat, params["mlp0_W"][:TRI],
                             r(params["bn_g"]), r(params["bn_b"]))
  zf = _zf(xflat, wf, mf, rgf, bef)

  hwp0, dinv8 = _prepb(hw0, degp)
  agg0 = edge_k(hwp0, src, dst, zeros64)
  hwp1, m1 = _mid(agg0, hwp0, dinv8, params["conv0_b"].reshape(1, HID),
                  params["conv1_W"])
  agg1 = edge_k(hwp1, src, dst, zeros64)

  out = _tail(
      agg1, hwp1, dinv8, params["conv1_b"].reshape(1, HID),
      zf, m1.reshape(NG, HID),
      r(params["bnh_g"]), r(params["bnh_b"]),
      params["mlp0_W"][TRI:], r(params["mlp0_b"]),
      r(params["mbn0_g"]), r(params["mbn0_b"]),
      params["mlp1_W"], r(params["mlp1_b"]),
      r(params["mbn1_g"]), r(params["mbn1_b"]),
      params["mlp2_W"], r(params["mlp2_b"]),
      r(params["mbn2_g"]), r(params["mbn2_b"]),
      params["mlp3_W"], r(params["mlp3_b"]))
  return out[:NG]
